# baseline (speedup 1.0000x reference)
import functools

import jax
import jax.numpy as jnp
from jax import lax
from jax.experimental import pallas as pl
from jax.experimental.pallas import tpu as pltpu
from jax.experimental.pallas import tpu_sc as plsc

_B = 16384
_MSG = 8
_V = 16
_D = _MSG * _V
_NC, _NS = 2, 16
_NW = _NC * _NS
_BPW = _B // _NW
_CHUNK = 128
_NCHUNK = _BPW // _CHUNK
_NWIN = _BPW * _MSG // 16

_LN2 = 0.6931471805599453


def _sc_body(msg_hbm, der_hbm, tab_hbm, out_hbm, idx_v, rows_v, msg_v, acc_v, sem):
    wid = lax.axis_index("s") * _NC + lax.axis_index("c")
    base = wid * _BPW

    for j in range(_NCHUNK):
        pltpu.sync_copy(der_hbm.at[pl.ds(base + j * _CHUNK, _CHUNK)], idx_v.at[j])
    cps = [
        pltpu.async_copy(
            tab_hbm.at[idx_v.at[j]], rows_v.at[pl.ds(j * _CHUNK, _CHUNK)],
            sem.at[j],
        )
        for j in range(_NCHUNK)
    ]
    pltpu.sync_copy(msg_hbm.at[pl.ds(base * _MSG, _BPW * _MSG)], msg_v)

    iota = lax.iota(jnp.int32, 16)
    row_off = lax.shift_right_logical(iota, 3)
    col_base = (iota & 7) * _V
    cols = [col_base + v for v in range(_V)]
    wpc = _NWIN // _NCHUNK

    def w_body(w, acc):
        msg = msg_v[pl.ds(w * 16, 16)]
        rb = row_off + w * 2
        tgt = plsc.load_gather(rows_v, [rb, col_base + msg])
        es = [jnp.exp(plsc.load_gather(rows_v, [rb, cols[v]])) for v in range(_V)]
        while len(es) > 1:
            es = [a + b for a, b in zip(es[::2], es[1::2])]
        s = es[0]
        bits = lax.bitcast_convert_type(s, jnp.int32)
        e = lax.shift_right_logical(bits, 23) - 127
        m = lax.bitcast_convert_type(
            (bits & 0x007FFFFF) | 0x3F800000, jnp.float32
        )
        r = (m - 1.0) / (m + 1.0)
        r2 = r * r
        lnm = r * (2.0 + r2 * (0.66666667 + r2 * (0.4 + r2 * 0.28571429)))
        logs = e.astype(jnp.float32) * _LN2 + lnm
        return acc + (logs - tgt)

    acc = jnp.zeros((16,), jnp.float32)
    for j in range(_NCHUNK):
        cps[j].wait()
        acc = plsc.parallel_loop(j * wpc, (j + 1) * wpc, unroll=2, carry=acc)(
            w_body
        )
    acc_v[...] = acc
    pltpu.sync_copy(acc_v, out_hbm.at[pl.ds(wid * 16, 16)])


_sc_kernel = functools.partial(
    pl.kernel,
    out_type=jax.ShapeDtypeStruct((_NW * 16,), jnp.float32),
    mesh=plsc.VectorSubcoreMesh(core_axis_name="c", subcore_axis_name="s"),
    compiler_params=pltpu.CompilerParams(needs_layout_passes=False),
    scratch_types=[
        pltpu.VMEM((_NCHUNK, _CHUNK), jnp.int32),
        pltpu.VMEM((_BPW, _D), jnp.float32),
        pltpu.VMEM((_BPW * _MSG,), jnp.int32),
        pltpu.VMEM((16,), jnp.float32),
        pltpu.SemaphoreType.DMA((_NCHUNK,)),
    ],
)(_sc_body)


def _reduce_body(p_ref, o_ref):
    o_ref[0, 0] = jnp.sum(p_ref[...]) * (1.0 / (_B * _MSG))


def _tc_reduce(partials):
    out = pl.pallas_call(
        _reduce_body,
        out_shape=jax.ShapeDtypeStruct((1, 1), jnp.float32),
        out_specs=pl.BlockSpec(memory_space=pltpu.SMEM),
    )(partials.reshape(4, 128))
    return out[0, 0]


def kernel(messages, derivations, emb_weight):
    partials = _sc_kernel(messages.reshape(-1), derivations, emb_weight)
    return _tc_reduce(partials)


# speedup vs baseline: 1.1234x; 1.1234x over previous
import functools

import jax
import jax.numpy as jnp
from jax import lax
from jax.experimental import pallas as pl
from jax.experimental.pallas import tpu as pltpu
from jax.experimental.pallas import tpu_sc as plsc

_B = 16384
_MSG = 8
_V = 16
_D = _MSG * _V
_NC, _NS = 2, 16
_NW = _NC * _NS
_BPW = _B // _NW
_CHUNK = 128
_NCHUNK = _BPW // _CHUNK
_NWIN = _BPW * _MSG // 16

_LN2 = 0.6931471805599453


def _sc_body(msg_hbm, der_hbm, tab_hbm, out_hbm, idx_v, rows_v, msg_v, acc_v, sem):
    wid = lax.axis_index("s") * _NC + lax.axis_index("c")
    base = wid * _BPW

    for j in range(_NCHUNK):
        pltpu.sync_copy(der_hbm.at[pl.ds(base + j * _CHUNK, _CHUNK)], idx_v.at[j])
    cps = [
        pltpu.async_copy(
            tab_hbm.at[idx_v.at[j]], rows_v.at[pl.ds(j * _CHUNK, _CHUNK)],
            sem.at[j],
        )
        for j in range(_NCHUNK)
    ]
    pltpu.sync_copy(msg_hbm.at[pl.ds(base * _MSG, _BPW * _MSG)], msg_v)

    iota = lax.iota(jnp.int32, 16)
    row_off = lax.shift_right_logical(iota, 3)
    pos = iota & 7
    col_base = pos * _V
    cols = [col_base + ((iota + v) & 15) for v in range(_V)]
    wpc = _NWIN // _NCHUNK

    def w_body(w, acc):
        rb = row_off + w * 2
        msg = msg_v[pl.ds(w * 16, 16)]
        tgt = plsc.load_gather(rows_v, [rb, col_base + msg])
        es = [jnp.exp(plsc.load_gather(rows_v, [rb, cols[v]])) for v in range(_V)]
        while len(es) > 1:
            es = [a + b for a, b in zip(es[::2], es[1::2])]
        s = es[0]
        bits = lax.bitcast_convert_type(s, jnp.int32)
        e = lax.shift_right_logical(bits, 23) - 127
        m = lax.bitcast_convert_type(
            (bits & 0x007FFFFF) | 0x3F800000, jnp.float32
        )
        r = (m - 1.0) / (m + 1.0)
        r2 = r * r
        lnm = r * (2.0 + r2 * (0.66666667 + r2 * (0.4 + r2 * 0.28571429)))
        logs = e.astype(jnp.float32) * _LN2 + lnm
        return acc + (logs - tgt)

    acc = jnp.zeros((16,), jnp.float32)
    for j in range(_NCHUNK):
        cps[j].wait()
        acc = plsc.parallel_loop(j * wpc, (j + 1) * wpc, unroll=2, carry=acc)(
            w_body
        )
    acc_v[...] = acc
    pltpu.sync_copy(acc_v, out_hbm.at[pl.ds(wid * 16, 16)])


_sc_kernel = functools.partial(
    pl.kernel,
    out_type=jax.ShapeDtypeStruct((_NW * 16,), jnp.float32),
    mesh=plsc.VectorSubcoreMesh(core_axis_name="c", subcore_axis_name="s"),
    compiler_params=pltpu.CompilerParams(needs_layout_passes=False),
    scratch_types=[
        pltpu.VMEM((_NCHUNK, _CHUNK), jnp.int32),
        pltpu.VMEM((_BPW, _D), jnp.float32),
        pltpu.VMEM((_BPW * _MSG,), jnp.int32),
        pltpu.VMEM((16,), jnp.float32),
        pltpu.SemaphoreType.DMA((_NCHUNK,)),
    ],
)(_sc_body)


def _reduce_body(p_ref, o_ref):
    o_ref[0, 0] = jnp.sum(p_ref[...]) * (1.0 / (_B * _MSG))


def _tc_reduce(partials):
    out = pl.pallas_call(
        _reduce_body,
        out_shape=jax.ShapeDtypeStruct((1, 1), jnp.float32),
        out_specs=pl.BlockSpec(memory_space=pltpu.SMEM),
    )(partials.reshape(4, 128))
    return out[0, 0]


def kernel(messages, derivations, emb_weight):
    partials = _sc_kernel(messages.reshape(-1), derivations, emb_weight)
    return _tc_reduce(partials)


# speedup vs baseline: 1.1235x; 1.0001x over previous
import functools

import jax
import jax.numpy as jnp
from jax import lax
from jax.experimental import pallas as pl
from jax.experimental.pallas import tpu as pltpu
from jax.experimental.pallas import tpu_sc as plsc

_B = 16384
_MSG = 8
_V = 16
_D = _MSG * _V
_NC, _NS = 2, 16
_NW = _NC * _NS
_BPW = _B // _NW
_CHUNK = 128
_NCHUNK = _BPW // _CHUNK
_NWIN = _BPW * _MSG // 16

_LN2 = 0.6931471805599453


def _sc_body(msg_hbm, der_hbm, tab_hbm, out_hbm, idx_v, rows_v, msg_v, acc_v, sem):
    wid = lax.axis_index("s") * _NC + lax.axis_index("c")
    base = wid * _BPW

    for j in range(_NCHUNK):
        pltpu.sync_copy(der_hbm.at[pl.ds(base + j * _CHUNK, _CHUNK)], idx_v.at[j])
    cps = [
        pltpu.async_copy(
            tab_hbm.at[idx_v.at[j]], rows_v.at[pl.ds(j * _CHUNK, _CHUNK)],
            sem.at[j],
        )
        for j in range(_NCHUNK)
    ]
    pltpu.sync_copy(msg_hbm.at[pl.ds(wid * (_BPW * _MSG // 128), _BPW * _MSG // 128), :], msg_v)

    iota = lax.iota(jnp.int32, 16)
    row_off = lax.shift_right_logical(iota, 3)
    pos = iota & 7
    col_base = pos * _V
    cols = [col_base + ((iota + v) & 15) for v in range(_V)]
    wpc = _NWIN // _NCHUNK

    def w_body(w, acc):
        rb = row_off + w * 2
        msg = msg_v[w >> 3, pl.ds((w & 7) * 16, 16)]
        tgt = plsc.load_gather(rows_v, [rb, col_base + msg])
        es = [jnp.exp(plsc.load_gather(rows_v, [rb, cols[v]])) for v in range(_V)]
        while len(es) > 1:
            es = [a + b for a, b in zip(es[::2], es[1::2])]
        s = es[0]
        bits = lax.bitcast_convert_type(s, jnp.int32)
        e = lax.shift_right_logical(bits, 23) - 127
        m = lax.bitcast_convert_type(
            (bits & 0x007FFFFF) | 0x3F800000, jnp.float32
        )
        r = (m - 1.0) / (m + 1.0)
        r2 = r * r
        lnm = r * (2.0 + r2 * (0.66666667 + r2 * (0.4 + r2 * 0.28571429)))
        logs = e.astype(jnp.float32) * _LN2 + lnm
        return acc + (logs - tgt)

    acc = jnp.zeros((16,), jnp.float32)
    for j in range(_NCHUNK):
        cps[j].wait()
        acc = plsc.parallel_loop(j * wpc, (j + 1) * wpc, unroll=2, carry=acc)(
            w_body
        )
    acc_v[...] = acc
    pltpu.sync_copy(acc_v, out_hbm.at[pl.ds(wid * 16, 16)])


_sc_kernel = functools.partial(
    pl.kernel,
    out_type=jax.ShapeDtypeStruct((_NW * 16,), jnp.float32),
    mesh=plsc.VectorSubcoreMesh(core_axis_name="c", subcore_axis_name="s"),
    compiler_params=pltpu.CompilerParams(needs_layout_passes=False),
    scratch_types=[
        pltpu.VMEM((_NCHUNK, _CHUNK), jnp.int32),
        pltpu.VMEM((_BPW, _D), jnp.float32),
        pltpu.VMEM((_BPW * _MSG // 128, 128), jnp.int32),
        pltpu.VMEM((16,), jnp.float32),
        pltpu.SemaphoreType.DMA((_NCHUNK,)),
    ],
)(_sc_body)


def _reduce_body(p_ref, o_ref):
    o_ref[0, 0] = jnp.sum(p_ref[...]) * (1.0 / (_B * _MSG))


def _tc_reduce(partials):
    out = pl.pallas_call(
        _reduce_body,
        out_shape=jax.ShapeDtypeStruct((1, 1), jnp.float32),
        out_specs=pl.BlockSpec(memory_space=pltpu.SMEM),
    )(partials.reshape(4, 128))
    return out[0, 0]


def kernel(messages, derivations, emb_weight):
    partials = _sc_kernel(messages.reshape(_B * _MSG // 128, 128), derivations, emb_weight)
    return _tc_reduce(partials)


# speedup vs baseline: 1.2941x; 1.1518x over previous
import functools

import jax
import jax.numpy as jnp
from jax import lax
from jax.experimental import pallas as pl
from jax.experimental.pallas import tpu as pltpu
from jax.experimental.pallas import tpu_sc as plsc

_B = 16384
_MSG = 8
_V = 16
_D = _MSG * _V
_NC, _NS = 2, 16
_NW = _NC * _NS
_BPW = _B // _NW
_CHUNK = 128
_NCHUNK = _BPW // _CHUNK
_NWIN = _BPW * _MSG // 16

_LN2 = 0.6931471805599453


def _sc_body(
    msg_hbm, der_hbm, tab_hbm, out_hbm, idx_v, rows_v, msg_v, acc_v, sem, msem
):
    wid = lax.axis_index("s") * _NC + lax.axis_index("c")
    base = wid * _BPW

    for j in range(_NCHUNK):
        pltpu.sync_copy(der_hbm.at[pl.ds(base + j * _CHUNK, _CHUNK)], idx_v.at[j])
    cps = [
        pltpu.async_copy(
            tab_hbm.at[idx_v.at[j]], rows_v.at[pl.ds(j * _CHUNK, _CHUNK)],
            sem.at[j],
        )
        for j in range(_NCHUNK)
    ]
    mcps = {
        j: pltpu.async_copy(
            msg_hbm.at[pl.ds(base + j * _CHUNK, _CHUNK), :], msg_v.at[j % 2],
            msem.at[j % 2],
        )
        for j in range(2)
    }

    iota = lax.iota(jnp.int32, 16)
    row_off = lax.shift_right_logical(iota, 3)
    pos = iota & 7
    col_base = pos * _V
    cols = [col_base + ((iota + v) & 15) for v in range(_V)]
    wpc = _NWIN // _NCHUNK

    def make_w_body(j):
        def w_body(w, acc):
            rb = row_off + w * 2
            rbl = row_off + (w - j * wpc) * 2
            msg = plsc.load_gather(msg_v.at[j % 2], [rbl, pos])
            tgt = plsc.load_gather(rows_v, [rb, col_base + msg])
            es = [
                jnp.exp(plsc.load_gather(rows_v, [rb, cols[v]]))
                for v in range(_V)
            ]
            while len(es) > 1:
                es = [a + b for a, b in zip(es[::2], es[1::2])]
            s = es[0]
            bits = lax.bitcast_convert_type(s, jnp.int32)
            e = lax.shift_right_logical(bits, 23) - 127
            m = lax.bitcast_convert_type(
                (bits & 0x007FFFFF) | 0x3F800000, jnp.float32
            )
            r = (m - 1.0) / (m + 1.0)
            r2 = r * r
            lnm = r * (2.0 + r2 * (0.66666667 + r2 * (0.4 + r2 * 0.28571429)))
            logs = e.astype(jnp.float32) * _LN2 + lnm
            return acc + (logs - tgt)

        return w_body

    acc = jnp.zeros((16,), jnp.float32)
    for j in range(_NCHUNK):
        cps[j].wait()
        mcps[j].wait()
        acc = plsc.parallel_loop(j * wpc, (j + 1) * wpc, unroll=2, carry=acc)(
            make_w_body(j)
        )
        if j + 2 < _NCHUNK:
            mcps[j + 2] = pltpu.async_copy(
                msg_hbm.at[pl.ds(base + (j + 2) * _CHUNK, _CHUNK), :],
                msg_v.at[j % 2],
                msem.at[j % 2],
            )
    acc_v[...] = acc
    pltpu.sync_copy(acc_v, out_hbm.at[pl.ds(wid * 16, 16)])


_sc_kernel = functools.partial(
    pl.kernel,
    out_type=jax.ShapeDtypeStruct((_NW * 16,), jnp.float32),
    mesh=plsc.VectorSubcoreMesh(core_axis_name="c", subcore_axis_name="s"),
    compiler_params=pltpu.CompilerParams(needs_layout_passes=False),
    scratch_types=[
        pltpu.VMEM((_NCHUNK, _CHUNK), jnp.int32),
        pltpu.VMEM((_BPW, _D), jnp.float32),
        pltpu.VMEM((2, _CHUNK, _MSG), jnp.int32),
        pltpu.VMEM((16,), jnp.float32),
        pltpu.SemaphoreType.DMA((_NCHUNK,)),
        pltpu.SemaphoreType.DMA((2,)),
    ],
)(_sc_body)


def _reduce_body(p_ref, o_ref):
    o_ref[0, 0] = jnp.sum(p_ref[...]) * (1.0 / (_B * _MSG))


def _tc_reduce(partials):
    out = pl.pallas_call(
        _reduce_body,
        out_shape=jax.ShapeDtypeStruct((1, 1), jnp.float32),
        out_specs=pl.BlockSpec(memory_space=pltpu.SMEM),
    )(partials.reshape(4, 128))
    return out[0, 0]


def kernel(messages, derivations, emb_weight):
    partials = _sc_kernel(messages, derivations, emb_weight)
    return _tc_reduce(partials)


# speedup vs baseline: 1.3008x; 1.0052x over previous
import functools

import jax
import jax.numpy as jnp
from jax import lax
from jax.experimental import pallas as pl
from jax.experimental.pallas import tpu as pltpu
from jax.experimental.pallas import tpu_sc as plsc

_B = 16384
_MSG = 8
_V = 16
_D = _MSG * _V
_NC, _NS = 2, 16
_NW = _NC * _NS
_BPW = _B // _NW
_CHUNK = 128
_NCHUNK = _BPW // _CHUNK
_NWIN = _BPW * _MSG // 16

_LN2 = 0.6931471805599453


def _sc_body(
    msg_hbm, der_hbm, tab_hbm, out_hbm, idx_v, rows_v, msg_v, acc_v, sem, msem
):
    wid = lax.axis_index("s") * _NC + lax.axis_index("c")
    base = wid * _BPW

    for j in range(_NCHUNK):
        pltpu.sync_copy(der_hbm.at[pl.ds(base + j * _CHUNK, _CHUNK)], idx_v.at[j])
    cps = [
        pltpu.async_copy(
            tab_hbm.at[idx_v.at[j]], rows_v.at[pl.ds(j * _CHUNK, _CHUNK)],
            sem.at[j],
        )
        for j in range(_NCHUNK)
    ]
    mcps = {
        j: pltpu.async_copy(
            msg_hbm.at[pl.ds(base + j * _CHUNK, _CHUNK), :], msg_v.at[j % 2],
            msem.at[j % 2],
        )
        for j in range(2)
    }

    iota = lax.iota(jnp.int32, 16)
    row_off = lax.shift_right_logical(iota, 3)
    pos = iota & 7
    col_base = pos * _V
    cols = [col_base + ((iota + v) & 15) for v in range(_V)]
    wpc = _NWIN // _NCHUNK

    def make_w_body(j):
        def w_body(w, acc):
            rb = row_off + w * 2
            rbl = row_off + (w - j * wpc) * 2
            msg = plsc.load_gather(msg_v.at[j % 2], [rbl, pos])
            tgt = plsc.load_gather(rows_v, [rb, col_base + msg])
            es = [
                jnp.exp(plsc.load_gather(rows_v, [rb, cols[v]]))
                for v in range(_V)
            ]
            while len(es) > 1:
                es = [a + b for a, b in zip(es[::2], es[1::2])]
            s = es[0]
            bits = lax.bitcast_convert_type(s, jnp.int32)
            e = lax.shift_right_logical(bits, 23) - 127
            m = lax.bitcast_convert_type(
                (bits & 0x007FFFFF) | 0x3F800000, jnp.float32
            )
            r = (m - 1.0) / (m + 1.0)
            r2 = r * r
            lnm = r * (2.0 + r2 * (0.66666667 + r2 * (0.4 + r2 * 0.28571429)))
            logs = e.astype(jnp.float32) * _LN2 + lnm
            return acc + (logs - tgt)

        return w_body

    acc = jnp.zeros((16,), jnp.float32)
    for j in range(_NCHUNK):
        cps[j].wait()
        mcps[j].wait()
        acc = plsc.parallel_loop(j * wpc, (j + 1) * wpc, unroll=1, carry=acc)(
            make_w_body(j)
        )
        if j + 2 < _NCHUNK:
            mcps[j + 2] = pltpu.async_copy(
                msg_hbm.at[pl.ds(base + (j + 2) * _CHUNK, _CHUNK), :],
                msg_v.at[j % 2],
                msem.at[j % 2],
            )
    acc_v[...] = acc
    pltpu.sync_copy(acc_v, out_hbm.at[pl.ds(wid * 16, 16)])


_sc_kernel = functools.partial(
    pl.kernel,
    out_type=jax.ShapeDtypeStruct((_NW * 16,), jnp.float32),
    mesh=plsc.VectorSubcoreMesh(core_axis_name="c", subcore_axis_name="s"),
    compiler_params=pltpu.CompilerParams(needs_layout_passes=False),
    scratch_types=[
        pltpu.VMEM((_NCHUNK, _CHUNK), jnp.int32),
        pltpu.VMEM((_BPW, _D), jnp.float32),
        pltpu.VMEM((2, _CHUNK, _MSG), jnp.int32),
        pltpu.VMEM((16,), jnp.float32),
        pltpu.SemaphoreType.DMA((_NCHUNK,)),
        pltpu.SemaphoreType.DMA((2,)),
    ],
)(_sc_body)


def _reduce_body(p_ref, o_ref):
    o_ref[0, 0] = jnp.sum(p_ref[...]) * (1.0 / (_B * _MSG))


def _tc_reduce(partials):
    out = pl.pallas_call(
        _reduce_body,
        out_shape=jax.ShapeDtypeStruct((1, 1), jnp.float32),
        out_specs=pl.BlockSpec(memory_space=pltpu.SMEM),
    )(partials.reshape(4, 128))
    return out[0, 0]


def kernel(messages, derivations, emb_weight):
    partials = _sc_kernel(messages, derivations, emb_weight)
    return _tc_reduce(partials)


# speedup vs baseline: 1.4934x; 1.1481x over previous
import functools

import jax
import jax.numpy as jnp
from jax import lax
from jax.experimental import pallas as pl
from jax.experimental.pallas import tpu as pltpu
from jax.experimental.pallas import tpu_sc as plsc

_B = 16384
_MSG = 8
_V = 16
_D = _MSG * _V
_NC, _NS = 2, 16
_NW = _NC * _NS
_BPW = _B // _NW
_CHUNK = 128
_NCHUNK = _BPW // _CHUNK
_NWIN = _BPW * _MSG // 16

_LN2 = 0.6931471805599453


def _sc_body(msg_hbm, der_hbm, tab_hbm, out_hbm, idx_v, rows_v, msg_v, acc_v, sem):
    wid = lax.axis_index("s") * _NC + lax.axis_index("c")
    base = wid * _BPW

    pltpu.sync_copy(der_hbm.at[pl.ds(base, _BPW)], idx_v)
    pltpu.sync_copy(msg_hbm.at[pl.ds(base, _BPW)], msg_v)
    cps = [
        pltpu.async_copy(
            tab_hbm.at[idx_v.at[pl.ds(j * _CHUNK, _CHUNK)]],
            rows_v.at[pl.ds(j * _CHUNK, _CHUNK)],
            sem.at[j],
        )
        for j in range(_NCHUNK)
    ]

    iota = lax.iota(jnp.int32, 16)
    row_off = lax.shift_right_logical(iota, 3)
    pos4 = (iota & 7) * 4
    col_base = (iota & 7) * _V
    cols = [col_base + ((iota + v) & 15) for v in range(_V)]
    wpc = _NWIN // _NCHUNK

    def w_body(w, acc):
        rb = row_off + w * 2
        mp = plsc.load_gather(msg_v, [rb])
        msg = lax.shift_right_logical(mp, pos4) & 15
        tgt = plsc.load_gather(rows_v, [rb, col_base + msg])
        es = [
            jnp.exp(plsc.load_gather(rows_v, [rb, cols[v]])) for v in range(_V)
        ]
        while len(es) > 1:
            es = [a + b for a, b in zip(es[::2], es[1::2])]
        s = es[0]
        bits = lax.bitcast_convert_type(s, jnp.int32)
        e = lax.shift_right_logical(bits, 23) - 127
        m = lax.bitcast_convert_type(
            (bits & 0x007FFFFF) | 0x3F800000, jnp.float32
        )
        r = (m - 1.0) / (m + 1.0)
        r2 = r * r
        lnm = r * (2.0 + r2 * (0.66666667 + r2 * (0.4 + r2 * 0.28571429)))
        logs = e.astype(jnp.float32) * _LN2 + lnm
        return acc + (logs - tgt)

    acc = jnp.zeros((16,), jnp.float32)
    for j in range(_NCHUNK):
        cps[j].wait()
        acc = plsc.parallel_loop(j * wpc, (j + 1) * wpc, unroll=1, carry=acc)(
            w_body
        )
    acc_v[...] = acc
    pltpu.sync_copy(acc_v, out_hbm.at[pl.ds(wid * 16, 16)])


_sc_kernel = functools.partial(
    pl.kernel,
    out_type=jax.ShapeDtypeStruct((_NW * 16,), jnp.float32),
    mesh=plsc.VectorSubcoreMesh(core_axis_name="c", subcore_axis_name="s"),
    compiler_params=pltpu.CompilerParams(needs_layout_passes=False),
    scratch_types=[
        pltpu.VMEM((_BPW,), jnp.int32),
        pltpu.VMEM((_BPW, _D), jnp.float32),
        pltpu.VMEM((_BPW,), jnp.int32),
        pltpu.VMEM((16,), jnp.float32),
        pltpu.SemaphoreType.DMA((_NCHUNK,)),
    ],
)(_sc_body)


def _reduce_body(p_ref, o_ref):
    o_ref[0, 0] = jnp.sum(p_ref[...]) * (1.0 / (_B * _MSG))


def _tc_reduce(partials):
    out = pl.pallas_call(
        _reduce_body,
        out_shape=jax.ShapeDtypeStruct((1, 1), jnp.float32),
        out_specs=pl.BlockSpec(memory_space=pltpu.SMEM),
    )(partials.reshape(4, 128))
    return out[0, 0]


def kernel(messages, derivations, emb_weight):
    shifts = (jnp.arange(_MSG, dtype=jnp.uint32) * 4)[None, :]
    packed = jnp.sum(
        messages.astype(jnp.uint32) << shifts, axis=1, dtype=jnp.uint32
    )
    packed = lax.bitcast_convert_type(packed, jnp.int32)
    partials = _sc_kernel(packed, derivations, emb_weight)
    return _tc_reduce(partials)
